# baseline (speedup 1.0000x reference)
import functools

import jax
import jax.numpy as jnp
from jax import lax
from jax.experimental import pallas as pl
from jax.experimental.pallas import tpu as pltpu
from jax.experimental.pallas import tpu_sc as plsc

N = 10000
E = 160000
E2 = 163840
EPT = E2 // 16
NB = EPT // 128
RPT = 624
NC = 2
DC = 128
R = 1000

_HIGH = jax.lax.Precision.HIGHEST


def _mesh():
    return plsc.VectorSubcoreMesh(core_axis_name="c", subcore_axis_name="s")


def _make_deg_kernel():
    DEPTH = 8

    @functools.partial(
        pl.kernel,
        mesh=_mesh(),
        out_type=jax.ShapeDtypeStruct((N, DC), jnp.float32),
        scratch_types=[
            pltpu.VMEM((NB, 128), jnp.int32),
            pltpu.VMEM((128, DC), jnp.float32),
            pltpu.VMEM_SHARED((N + 8, DC), jnp.float32),
            pltpu.SemaphoreType.DMA,
        ],
    )
    def deg_kernel(dst_hbm, zeros_hbm, ones_hbm, out_hbm, didx, ones_v, acc,
                   sem):
        core = lax.axis_index("c")
        sub = lax.axis_index("s")

        @pl.when(core == 0)
        def _core0():
            @pl.when(sub == 0)
            def _init():
                pltpu.sync_copy(zeros_hbm, acc.at[pl.ds(0, N)])

            pltpu.sync_copy(ones_hbm, ones_v)
            pltpu.sync_copy(dst_hbm.at[sub], didx)
            plsc.subcore_barrier()

            for r in range(DEPTH):
                pltpu.async_copy(ones_v, acc.at[didx.at[r]], sem, add=True)

            def body(j, carry):
                pltpu.make_async_copy(
                    zeros_hbm.at[pl.ds(0, 128)], ones_v, sem).wait()
                pltpu.async_copy(
                    ones_v, acc.at[didx.at[j + DEPTH]], sem, add=True)
                return carry

            lax.fori_loop(0, NB - DEPTH, body, 0)
            for r in range(DEPTH):
                pltpu.make_async_copy(
                    zeros_hbm.at[pl.ds(0, 128)], ones_v, sem).wait()
            plsc.subcore_barrier()
            pltpu.sync_copy(acc.at[pl.ds(sub * RPT, RPT)],
                            out_hbm.at[pl.ds(sub * RPT, RPT)])

            @pl.when(sub == 0)
            def _tail():
                pltpu.sync_copy(acc.at[pl.ds(16 * RPT, N - 16 * RPT)],
                                out_hbm.at[pl.ds(16 * RPT, N - 16 * RPT)])

    return deg_kernel


def _make_agg_kernel(nchunk):
    outs = [jax.ShapeDtypeStruct((N, DC), jnp.float32)] * nchunk

    HB = NB // 2

    @functools.partial(
        pl.kernel,
        mesh=_mesh(),
        out_type=outs,
        scratch_types=[
            pltpu.VMEM((HB, 128), jnp.int32),
            pltpu.VMEM((HB, 128), jnp.int32),
            pltpu.VMEM((128, DC), jnp.float32),
            pltpu.VMEM((128, DC), jnp.float32),
            pltpu.VMEM_SHARED((N + 8, DC), jnp.float32),
            pltpu.SemaphoreType.DMA,
            pltpu.SemaphoreType.DMA,
            pltpu.SemaphoreType.DMA,
            pltpu.SemaphoreType.DMA,
        ],
    )
    def agg_kernel(*refs):
        u = refs[:nchunk]
        src_hbm, dst_hbm = refs[nchunk], refs[nchunk + 1]
        out = refs[nchunk + 2:2 * nchunk + 2]
        (sidx, didx, rows0, rows1, acc,
         gsem0, gsem1, ssem0, ssem1) = refs[2 * nchunk + 2:]
        core = lax.axis_index("c")
        sub = lax.axis_index("s")

        def _drain(dst, sem):
            pltpu.make_async_copy(u[0].at[pl.ds(0, 128)], dst, sem).wait()

        for ci in range(nchunk):
            def _chunk(ci=ci):
                @pl.when(sub == 0)
                def _init():
                    pltpu.sync_copy(u[ci], acc.at[pl.ds(0, N)])

                plsc.subcore_barrier()

                for h in range(2):
                    pltpu.sync_copy(src_hbm.at[sub, pl.ds(h * HB, HB)], sidx)
                    pltpu.sync_copy(dst_hbm.at[sub, pl.ds(h * HB, HB)], didx)

                    pltpu.async_copy(u[ci].at[sidx.at[0]], rows0, gsem0)
                    pltpu.async_copy(u[ci].at[sidx.at[1]], rows1, gsem1)

                    def body(j, carry):
                        b0 = 2 * j
                        _drain(rows0, gsem0)
                        pltpu.async_copy(
                            rows0, acc.at[didx.at[b0]], ssem0, add=True)
                        _drain(rows1, gsem1)
                        pltpu.async_copy(
                            rows1, acc.at[didx.at[b0 + 1]], ssem1, add=True)
                        _drain(rows0, ssem0)

                        @pl.when(b0 + 2 < HB)
                        def _g0():
                            pltpu.async_copy(
                                u[ci].at[sidx.at[b0 + 2]], rows0, gsem0)

                        _drain(rows1, ssem1)

                        @pl.when(b0 + 3 < HB)
                        def _g1():
                            pltpu.async_copy(
                                u[ci].at[sidx.at[b0 + 3]], rows1, gsem1)

                        return carry

                    lax.fori_loop(0, HB // 2, body, 0)
                plsc.subcore_barrier()
                pltpu.sync_copy(acc.at[pl.ds(sub * RPT, RPT)],
                                out[ci].at[pl.ds(sub * RPT, RPT)])

                @pl.when(sub == 0)
                def _tail():
                    pltpu.sync_copy(acc.at[pl.ds(16 * RPT, N - 16 * RPT)],
                                    out[ci].at[pl.ds(16 * RPT, N - 16 * RPT)])

                plsc.subcore_barrier()

            pl.when(core == (ci % NC))(_chunk)

    return agg_kernel


def _dis(deg_blk):
    return jax.lax.rsqrt(deg_blk[:, :1] + 1.0)


def _scale_x_body(x_ref, deg_ref, o0_ref, o1_ref):
    d = _dis(deg_ref[...])
    u = x_ref[...] * d
    o0_ref[...] = u[:, :DC]
    o1_ref[...] = u[:, DC:]


def _scale_x(x, deg16):
    return pl.pallas_call(
        _scale_x_body,
        grid=(N // R,),
        in_specs=[
            pl.BlockSpec((R, 2 * DC), lambda r: (r, 0)),
            pl.BlockSpec((R, DC), lambda r: (r, 0)),
        ],
        out_specs=[pl.BlockSpec((R, DC), lambda r: (r, 0))] * 2,
        out_shape=[jax.ShapeDtypeStruct((N, DC), jnp.float32)] * 2,
    )(x, deg16)


def _gcn_mid_body(nc_in, *refs):
    v = refs[:nc_in]
    deg_ref, w_ref, b_ref = refs[nc_in:nc_in + 3]
    outs = refs[nc_in + 3:]
    d = _dis(deg_ref[...])
    vv = jnp.concatenate([r[...] for r in v], axis=1) * d
    h = jnp.dot(vv, w_ref[...], precision=_HIGH) + b_ref[0]
    h = jnp.maximum(h, 0.0) * d
    for k, o in enumerate(outs):
        o[...] = h[:, k * DC:(k + 1) * DC]


def _gcn_mid(v_chunks, deg16, w, b2d):
    nc_in = len(v_chunks)
    k_dim = nc_in * DC
    return pl.pallas_call(
        functools.partial(_gcn_mid_body, nc_in),
        grid=(N // R,),
        in_specs=(
            [pl.BlockSpec((R, DC), lambda r: (r, 0))] * nc_in
            + [
                pl.BlockSpec((R, DC), lambda r: (r, 0)),
                pl.BlockSpec((k_dim, 512), lambda r: (0, 0)),
                pl.BlockSpec((1, 512), lambda r: (0, 0)),
            ]
        ),
        out_specs=[pl.BlockSpec((R, DC), lambda r: (r, 0))] * 4,
        out_shape=[jax.ShapeDtypeStruct((N, DC), jnp.float32)] * 4,
    )(*v_chunks, deg16, w, b2d)


def _gcn_final_body(*refs):
    v = refs[:4]
    deg_ref, w3_ref, b3_ref, wc_ref, o0_ref, o1_ref = refs[4:]
    d = _dis(deg_ref[...])
    vv = jnp.concatenate([r[...] for r in v], axis=1) * d
    h = jnp.dot(vv, w3_ref[...], precision=_HIGH) + b3_ref[0]
    h = jnp.maximum(h, 0.0)
    z = jnp.dot(h, wc_ref[...], precision=_HIGH) * d
    o0_ref[...] = z[:, :DC]
    o1_ref[...] = z[:, DC:]


def _gcn_final(v_chunks, deg16, w3, b3_2d, wc):
    return pl.pallas_call(
        _gcn_final_body,
        grid=(N // R,),
        in_specs=(
            [pl.BlockSpec((R, DC), lambda r: (r, 0))] * 4
            + [
                pl.BlockSpec((R, DC), lambda r: (r, 0)),
                pl.BlockSpec((512, 512), lambda r: (0, 0)),
                pl.BlockSpec((1, 512), lambda r: (0, 0)),
                pl.BlockSpec((512, 2 * DC), lambda r: (0, 0)),
            ]
        ),
        out_specs=[pl.BlockSpec((R, DC), lambda r: (r, 0))] * 2,
        out_shape=[jax.ShapeDtypeStruct((N, DC), jnp.float32)] * 2,
    )(*v_chunks, deg16, w3, b3_2d, wc)


def _finish_body(v0_ref, v1_ref, deg_ref, bmu_ref, blv_ref, mu_ref, lv_ref):
    d = _dis(deg_ref[...])
    mu_ref[...] = v0_ref[...] * d + bmu_ref[0]
    lv_ref[...] = v1_ref[...] * d + blv_ref[0]


def _finish(v_chunks, deg16, bmu2d, blv2d):
    return pl.pallas_call(
        _finish_body,
        grid=(N // R,),
        in_specs=[
            pl.BlockSpec((R, DC), lambda r: (r, 0)),
            pl.BlockSpec((R, DC), lambda r: (r, 0)),
            pl.BlockSpec((R, DC), lambda r: (r, 0)),
            pl.BlockSpec((1, DC), lambda r: (0, 0)),
            pl.BlockSpec((1, DC), lambda r: (0, 0)),
        ],
        out_specs=[pl.BlockSpec((R, DC), lambda r: (r, 0))] * 2,
        out_shape=[jax.ShapeDtypeStruct((N, DC), jnp.float32)] * 2,
    )(*v_chunks, deg16, bmu2d, blv2d)


def kernel(x, edge_index, W1, b1, W2, b2, W3, b3, W_mu, b_mu, W_lv, b_lv):
    src = edge_index[0].astype(jnp.int32)
    dst = edge_index[1].astype(jnp.int32)
    pad = jnp.arange(E2 - E, dtype=jnp.int32)
    src_p = jnp.concatenate([src, pad % N])
    dst_p = jnp.concatenate([dst, N + (pad % 8)])

    src3 = src_p.reshape(16, NB, 128)
    dst3 = dst_p.reshape(16, NB, 128)

    zeros16 = jnp.zeros((N, DC), jnp.float32)
    ones16 = jnp.ones((128, DC), jnp.float32)
    deg16 = _make_deg_kernel()(dst3, zeros16, ones16)

    agg2 = _make_agg_kernel(2)
    agg4 = _make_agg_kernel(4)
    u0 = _scale_x(x, deg16)
    v1 = agg2(u0[0], u0[1], src3, dst3)
    u1 = _gcn_mid(v1, deg16, W1, b1.reshape(1, 512))
    v2 = agg4(*u1, src3, dst3)
    u2 = _gcn_mid(v2, deg16, W2, b2.reshape(1, 512))
    v3 = agg4(*u2, src3, dst3)
    wc = jnp.concatenate([W_mu, W_lv], axis=1)
    u3 = _gcn_final(v3, deg16, W3, b3.reshape(1, 512), wc)
    v4 = agg2(u3[0], u3[1], src3, dst3)
    mu, lv = _finish(v4, deg16, b_mu.reshape(1, DC), b_lv.reshape(1, DC))
    return (mu, lv)


# speedup vs baseline: 1.1436x; 1.1436x over previous
import functools

import jax
import jax.numpy as jnp
from jax import lax
from jax.experimental import pallas as pl
from jax.experimental.pallas import tpu as pltpu
from jax.experimental.pallas import tpu_sc as plsc

N = 10000
E = 160000
E2 = 163840
EPT = E2 // 16
NB = EPT // 128
RPT = 624
NC = 2
DC = 128
R = 1000

_HIGH = jax.lax.Precision.HIGHEST


def _mesh():
    return plsc.VectorSubcoreMesh(core_axis_name="c", subcore_axis_name="s")


def _make_deg_kernel():
    DEPTH = 8

    @functools.partial(
        pl.kernel,
        mesh=_mesh(),
        out_type=jax.ShapeDtypeStruct((N, DC), jnp.float32),
        scratch_types=[
            pltpu.VMEM((NB, 128), jnp.int32),
            pltpu.VMEM((128, DC), jnp.float32),
            pltpu.VMEM_SHARED((N + 8, DC), jnp.float32),
            pltpu.SemaphoreType.DMA,
        ],
    )
    def deg_kernel(dst_hbm, zeros_hbm, ones_hbm, out_hbm, didx, ones_v, acc,
                   sem):
        core = lax.axis_index("c")
        sub = lax.axis_index("s")

        @pl.when(core == 0)
        def _core0():
            @pl.when(sub == 0)
            def _init():
                pltpu.sync_copy(zeros_hbm, acc.at[pl.ds(0, N)])

            pltpu.sync_copy(ones_hbm, ones_v)
            pltpu.sync_copy(dst_hbm.at[sub], didx)
            plsc.subcore_barrier()

            for r in range(DEPTH):
                pltpu.async_copy(ones_v, acc.at[didx.at[r]], sem, add=True)

            def body(j, carry):
                pltpu.make_async_copy(
                    zeros_hbm.at[pl.ds(0, 128)], ones_v, sem).wait()
                pltpu.async_copy(
                    ones_v, acc.at[didx.at[j + DEPTH]], sem, add=True)
                return carry

            lax.fori_loop(0, NB - DEPTH, body, 0)
            for r in range(DEPTH):
                pltpu.make_async_copy(
                    zeros_hbm.at[pl.ds(0, 128)], ones_v, sem).wait()
            plsc.subcore_barrier()
            pltpu.sync_copy(acc.at[pl.ds(sub * RPT, RPT)],
                            out_hbm.at[pl.ds(sub * RPT, RPT)])

            @pl.when(sub == 0)
            def _tail():
                pltpu.sync_copy(acc.at[pl.ds(16 * RPT, N - 16 * RPT)],
                                out_hbm.at[pl.ds(16 * RPT, N - 16 * RPT)])

    return deg_kernel


def _make_agg_kernel(nchunk):
    outs = [jax.ShapeDtypeStruct((N, DC), jnp.float32)] * nchunk

    BE = 64
    NB2 = EPT // BE
    HB = NB2 // 4
    NS = 4

    @functools.partial(
        pl.kernel,
        mesh=_mesh(),
        out_type=outs,
        scratch_types=[
            pltpu.VMEM((HB, BE), jnp.int32),
            pltpu.VMEM((HB, BE), jnp.int32),
            [pltpu.VMEM((BE, DC), jnp.float32)] * NS,
            pltpu.VMEM_SHARED((N + 8, DC), jnp.float32),
            [pltpu.SemaphoreType.DMA] * NS,
            [pltpu.SemaphoreType.DMA] * NS,
        ],
    )
    def agg_kernel(*refs):
        u = refs[:nchunk]
        src_hbm, dst_hbm = refs[nchunk], refs[nchunk + 1]
        out = refs[nchunk + 2:2 * nchunk + 2]
        sidx, didx, rows, acc, gsem, ssem = refs[2 * nchunk + 2:]
        core = lax.axis_index("c")
        sub = lax.axis_index("s")

        def _drain(dst, sem):
            pltpu.make_async_copy(u[0].at[pl.ds(0, BE)], dst, sem).wait()

        for ci in range(nchunk):
            def _chunk(ci=ci):
                @pl.when(sub == 0)
                def _init():
                    pltpu.sync_copy(u[ci], acc.at[pl.ds(0, N)])

                plsc.subcore_barrier()

                for h in range(4):
                    pltpu.sync_copy(src_hbm.at[sub, pl.ds(h * HB, HB)], sidx)
                    pltpu.sync_copy(dst_hbm.at[sub, pl.ds(h * HB, HB)], didx)

                    for r in range(NS):
                        pltpu.async_copy(
                            u[ci].at[sidx.at[r]], rows[r], gsem[r])

                    def body(j, carry):
                        b = NS * j
                        for r in range(NS):
                            _drain(rows[r], gsem[r])
                            pltpu.async_copy(rows[r], acc.at[didx.at[b + r]],
                                             ssem[r], add=True)
                        for r in range(NS):
                            _drain(rows[r], ssem[r])

                            @pl.when(b + NS + r < HB)
                            def _g(r=r):
                                pltpu.async_copy(u[ci].at[sidx.at[b + NS + r]],
                                                 rows[r], gsem[r])

                        return carry

                    lax.fori_loop(0, HB // NS, body, 0)
                plsc.subcore_barrier()
                pltpu.sync_copy(acc.at[pl.ds(sub * RPT, RPT)],
                                out[ci].at[pl.ds(sub * RPT, RPT)])

                @pl.when(sub == 0)
                def _tail():
                    pltpu.sync_copy(acc.at[pl.ds(16 * RPT, N - 16 * RPT)],
                                    out[ci].at[pl.ds(16 * RPT, N - 16 * RPT)])

                plsc.subcore_barrier()

            pl.when(core == (ci % NC))(_chunk)

    return agg_kernel


def _dis(deg_blk):
    return jax.lax.rsqrt(deg_blk[:, :1] + 1.0)


def _scale_x_body(x_ref, deg_ref, o0_ref, o1_ref):
    d = _dis(deg_ref[...])
    u = x_ref[...] * d
    o0_ref[...] = u[:, :DC]
    o1_ref[...] = u[:, DC:]


def _scale_x(x, deg16):
    return pl.pallas_call(
        _scale_x_body,
        grid=(N // R,),
        in_specs=[
            pl.BlockSpec((R, 2 * DC), lambda r: (r, 0)),
            pl.BlockSpec((R, DC), lambda r: (r, 0)),
        ],
        out_specs=[pl.BlockSpec((R, DC), lambda r: (r, 0))] * 2,
        out_shape=[jax.ShapeDtypeStruct((N, DC), jnp.float32)] * 2,
    )(x, deg16)


def _gcn_mid_body(nc_in, *refs):
    v = refs[:nc_in]
    deg_ref, w_ref, b_ref = refs[nc_in:nc_in + 3]
    outs = refs[nc_in + 3:]
    d = _dis(deg_ref[...])
    vv = jnp.concatenate([r[...] for r in v], axis=1) * d
    h = jnp.dot(vv, w_ref[...], precision=_HIGH) + b_ref[0]
    h = jnp.maximum(h, 0.0) * d
    for k, o in enumerate(outs):
        o[...] = h[:, k * DC:(k + 1) * DC]


def _gcn_mid(v_chunks, deg16, w, b2d):
    nc_in = len(v_chunks)
    k_dim = nc_in * DC
    return pl.pallas_call(
        functools.partial(_gcn_mid_body, nc_in),
        grid=(N // R,),
        in_specs=(
            [pl.BlockSpec((R, DC), lambda r: (r, 0))] * nc_in
            + [
                pl.BlockSpec((R, DC), lambda r: (r, 0)),
                pl.BlockSpec((k_dim, 512), lambda r: (0, 0)),
                pl.BlockSpec((1, 512), lambda r: (0, 0)),
            ]
        ),
        out_specs=[pl.BlockSpec((R, DC), lambda r: (r, 0))] * 4,
        out_shape=[jax.ShapeDtypeStruct((N, DC), jnp.float32)] * 4,
    )(*v_chunks, deg16, w, b2d)


def _gcn_final_body(*refs):
    v = refs[:4]
    deg_ref, w3_ref, b3_ref, wc_ref, o0_ref, o1_ref = refs[4:]
    d = _dis(deg_ref[...])
    vv = jnp.concatenate([r[...] for r in v], axis=1) * d
    h = jnp.dot(vv, w3_ref[...], precision=_HIGH) + b3_ref[0]
    h = jnp.maximum(h, 0.0)
    z = jnp.dot(h, wc_ref[...], precision=_HIGH) * d
    o0_ref[...] = z[:, :DC]
    o1_ref[...] = z[:, DC:]


def _gcn_final(v_chunks, deg16, w3, b3_2d, wc):
    return pl.pallas_call(
        _gcn_final_body,
        grid=(N // R,),
        in_specs=(
            [pl.BlockSpec((R, DC), lambda r: (r, 0))] * 4
            + [
                pl.BlockSpec((R, DC), lambda r: (r, 0)),
                pl.BlockSpec((512, 512), lambda r: (0, 0)),
                pl.BlockSpec((1, 512), lambda r: (0, 0)),
                pl.BlockSpec((512, 2 * DC), lambda r: (0, 0)),
            ]
        ),
        out_specs=[pl.BlockSpec((R, DC), lambda r: (r, 0))] * 2,
        out_shape=[jax.ShapeDtypeStruct((N, DC), jnp.float32)] * 2,
    )(*v_chunks, deg16, w3, b3_2d, wc)


def _finish_body(v0_ref, v1_ref, deg_ref, bmu_ref, blv_ref, mu_ref, lv_ref):
    d = _dis(deg_ref[...])
    mu_ref[...] = v0_ref[...] * d + bmu_ref[0]
    lv_ref[...] = v1_ref[...] * d + blv_ref[0]


def _finish(v_chunks, deg16, bmu2d, blv2d):
    return pl.pallas_call(
        _finish_body,
        grid=(N // R,),
        in_specs=[
            pl.BlockSpec((R, DC), lambda r: (r, 0)),
            pl.BlockSpec((R, DC), lambda r: (r, 0)),
            pl.BlockSpec((R, DC), lambda r: (r, 0)),
            pl.BlockSpec((1, DC), lambda r: (0, 0)),
            pl.BlockSpec((1, DC), lambda r: (0, 0)),
        ],
        out_specs=[pl.BlockSpec((R, DC), lambda r: (r, 0))] * 2,
        out_shape=[jax.ShapeDtypeStruct((N, DC), jnp.float32)] * 2,
    )(*v_chunks, deg16, bmu2d, blv2d)


def kernel(x, edge_index, W1, b1, W2, b2, W3, b3, W_mu, b_mu, W_lv, b_lv):
    src = edge_index[0].astype(jnp.int32)
    dst = edge_index[1].astype(jnp.int32)
    pad = jnp.arange(E2 - E, dtype=jnp.int32)
    src_p = jnp.concatenate([src, pad % N])
    dst_p = jnp.concatenate([dst, N + (pad % 8)])

    src3 = src_p.reshape(16, EPT // 64, 64)
    dst3 = dst_p.reshape(16, EPT // 64, 64)
    dst3deg = dst_p.reshape(16, NB, 128)

    zeros16 = jnp.zeros((N, DC), jnp.float32)
    ones16 = jnp.ones((128, DC), jnp.float32)
    deg16 = _make_deg_kernel()(dst3deg, zeros16, ones16)

    agg2 = _make_agg_kernel(2)
    agg4 = _make_agg_kernel(4)
    u0 = _scale_x(x, deg16)
    v1 = agg2(u0[0], u0[1], src3, dst3)
    u1 = _gcn_mid(v1, deg16, W1, b1.reshape(1, 512))
    v2 = agg4(*u1, src3, dst3)
    u2 = _gcn_mid(v2, deg16, W2, b2.reshape(1, 512))
    v3 = agg4(*u2, src3, dst3)
    wc = jnp.concatenate([W_mu, W_lv], axis=1)
    u3 = _gcn_final(v3, deg16, W3, b3.reshape(1, 512), wc)
    v4 = agg2(u3[0], u3[1], src3, dst3)
    mu, lv = _finish(v4, deg16, b_mu.reshape(1, DC), b_lv.reshape(1, DC))
    return (mu, lv)


# speedup vs baseline: 1.2050x; 1.0537x over previous
import functools

import jax
import jax.numpy as jnp
from jax import lax
from jax.experimental import pallas as pl
from jax.experimental.pallas import tpu as pltpu
from jax.experimental.pallas import tpu_sc as plsc

N = 10000
E = 160000
E2 = 163840
EPT = E2 // 16
NB = EPT // 128
RPT = 624
NC = 2
DC = 128
R = 1000

_HIGH = jax.lax.Precision.HIGHEST


def _mesh():
    return plsc.VectorSubcoreMesh(core_axis_name="c", subcore_axis_name="s")


def _make_deg_kernel():
    DEPTH = 8

    @functools.partial(
        pl.kernel,
        mesh=_mesh(),
        out_type=jax.ShapeDtypeStruct((N, DC), jnp.float32),
        scratch_types=[
            pltpu.VMEM((NB, 128), jnp.int32),
            pltpu.VMEM((128, DC), jnp.float32),
            pltpu.VMEM_SHARED((N + 8, DC), jnp.float32),
            pltpu.SemaphoreType.DMA,
        ],
    )
    def deg_kernel(dst_hbm, zeros_hbm, ones_hbm, out_hbm, didx, ones_v, acc,
                   sem):
        core = lax.axis_index("c")
        sub = lax.axis_index("s")

        @pl.when(core == 0)
        def _core0():
            @pl.when(sub == 0)
            def _init():
                pltpu.sync_copy(zeros_hbm, acc.at[pl.ds(0, N)])

            pltpu.sync_copy(ones_hbm, ones_v)
            pltpu.sync_copy(dst_hbm.at[sub], didx)
            plsc.subcore_barrier()

            for r in range(DEPTH):
                pltpu.async_copy(ones_v, acc.at[didx.at[r]], sem, add=True)

            def body(j, carry):
                pltpu.make_async_copy(
                    zeros_hbm.at[pl.ds(0, 128)], ones_v, sem).wait()
                pltpu.async_copy(
                    ones_v, acc.at[didx.at[j + DEPTH]], sem, add=True)
                return carry

            lax.fori_loop(0, NB - DEPTH, body, 0)
            for r in range(DEPTH):
                pltpu.make_async_copy(
                    zeros_hbm.at[pl.ds(0, 128)], ones_v, sem).wait()
            plsc.subcore_barrier()
            pltpu.sync_copy(acc.at[pl.ds(sub * RPT, RPT)],
                            out_hbm.at[pl.ds(sub * RPT, RPT)])

            @pl.when(sub == 0)
            def _tail():
                pltpu.sync_copy(acc.at[pl.ds(16 * RPT, N - 16 * RPT)],
                                out_hbm.at[pl.ds(16 * RPT, N - 16 * RPT)])

    return deg_kernel


def _make_agg_kernel(nchunk):
    outs = [jax.ShapeDtypeStruct((N, DC), jnp.float32)] * nchunk

    HB = NB // 2

    @functools.partial(
        pl.kernel,
        mesh=_mesh(),
        out_type=outs,
        scratch_types=[
            pltpu.VMEM((HB, 128), jnp.int32),
            pltpu.VMEM((HB, 128), jnp.int32),
            pltpu.VMEM((128, DC), jnp.float32),
            pltpu.VMEM((128, DC), jnp.float32),
            pltpu.VMEM_SHARED((N + 8, DC), jnp.float32),
            pltpu.SemaphoreType.DMA,
            pltpu.SemaphoreType.DMA,
        ],
    )
    def agg_kernel(*refs):
        u = refs[:nchunk]
        src_hbm, dst_hbm = refs[nchunk], refs[nchunk + 1]
        out = refs[nchunk + 2:2 * nchunk + 2]
        sidx, didx, rows0, rows1, acc, sem0, sem1 = refs[2 * nchunk + 2:]
        core = lax.axis_index("c")
        sub = lax.axis_index("s")

        for ci in range(nchunk):
            def _chunk(ci=ci):
                @pl.when(sub == 0)
                def _init():
                    pltpu.sync_copy(u[ci], acc.at[pl.ds(0, N)])

                plsc.subcore_barrier()

                for h in range(2):
                    pltpu.sync_copy(src_hbm.at[sub, pl.ds(h * HB, HB)], sidx)
                    pltpu.sync_copy(dst_hbm.at[sub, pl.ds(h * HB, HB)], didx)

                    pltpu.async_copy(u[ci].at[sidx.at[0]], rows0, sem0)

                    def body(j, carry):
                        b0 = 2 * j
                        pltpu.make_async_copy(
                            u[ci].at[pl.ds(0, 128)], rows0, sem0).wait()
                        pltpu.async_copy(
                            u[ci].at[sidx.at[b0 + 1]], rows1, sem1)
                        pltpu.sync_copy(rows0, acc.at[didx.at[b0]], add=True)

                        @pl.when(b0 + 2 < HB)
                        def _next():
                            pltpu.async_copy(
                                u[ci].at[sidx.at[b0 + 2]], rows0, sem0)

                        pltpu.make_async_copy(
                            u[ci].at[pl.ds(0, 128)], rows1, sem1).wait()
                        pltpu.sync_copy(
                            rows1, acc.at[didx.at[b0 + 1]], add=True)
                        return carry

                    lax.fori_loop(0, HB // 2, body, 0)
                plsc.subcore_barrier()
                pltpu.sync_copy(acc.at[pl.ds(sub * RPT, RPT)],
                                out[ci].at[pl.ds(sub * RPT, RPT)])

                @pl.when(sub == 0)
                def _tail():
                    pltpu.sync_copy(acc.at[pl.ds(16 * RPT, N - 16 * RPT)],
                                    out[ci].at[pl.ds(16 * RPT, N - 16 * RPT)])

                plsc.subcore_barrier()

            pl.when(core == (ci % NC))(_chunk)

    return agg_kernel


def _dis(deg_blk):
    return jax.lax.rsqrt(deg_blk[:, :1] + 1.0)


def _scale_x_body(x_ref, deg_ref, o0_ref, o1_ref):
    d = _dis(deg_ref[...])
    u = x_ref[...] * d
    o0_ref[...] = u[:, :DC]
    o1_ref[...] = u[:, DC:]


def _scale_x(x, deg16):
    return pl.pallas_call(
        _scale_x_body,
        grid=(N // R,),
        in_specs=[
            pl.BlockSpec((R, 2 * DC), lambda r: (r, 0)),
            pl.BlockSpec((R, DC), lambda r: (r, 0)),
        ],
        out_specs=[pl.BlockSpec((R, DC), lambda r: (r, 0))] * 2,
        out_shape=[jax.ShapeDtypeStruct((N, DC), jnp.float32)] * 2,
    )(x, deg16)


def _gcn_mid_body(nc_in, *refs):
    v = refs[:nc_in]
    deg_ref, w_ref, b_ref = refs[nc_in:nc_in + 3]
    outs = refs[nc_in + 3:]
    d = _dis(deg_ref[...])
    vv = jnp.concatenate([r[...] for r in v], axis=1) * d
    h = jnp.dot(vv, w_ref[...], precision=_HIGH) + b_ref[0]
    h = jnp.maximum(h, 0.0) * d
    for k, o in enumerate(outs):
        o[...] = h[:, k * DC:(k + 1) * DC]


def _gcn_mid(v_chunks, deg16, w, b2d):
    nc_in = len(v_chunks)
    k_dim = nc_in * DC
    return pl.pallas_call(
        functools.partial(_gcn_mid_body, nc_in),
        grid=(N // R,),
        in_specs=(
            [pl.BlockSpec((R, DC), lambda r: (r, 0))] * nc_in
            + [
                pl.BlockSpec((R, DC), lambda r: (r, 0)),
                pl.BlockSpec((k_dim, 512), lambda r: (0, 0)),
                pl.BlockSpec((1, 512), lambda r: (0, 0)),
            ]
        ),
        out_specs=[pl.BlockSpec((R, DC), lambda r: (r, 0))] * 4,
        out_shape=[jax.ShapeDtypeStruct((N, DC), jnp.float32)] * 4,
    )(*v_chunks, deg16, w, b2d)


def _gcn_final_body(*refs):
    v = refs[:4]
    deg_ref, w3_ref, b3_ref, wc_ref, o0_ref, o1_ref = refs[4:]
    d = _dis(deg_ref[...])
    vv = jnp.concatenate([r[...] for r in v], axis=1) * d
    h = jnp.dot(vv, w3_ref[...], precision=_HIGH) + b3_ref[0]
    h = jnp.maximum(h, 0.0)
    z = jnp.dot(h, wc_ref[...], precision=_HIGH) * d
    o0_ref[...] = z[:, :DC]
    o1_ref[...] = z[:, DC:]


def _gcn_final(v_chunks, deg16, w3, b3_2d, wc):
    return pl.pallas_call(
        _gcn_final_body,
        grid=(N // R,),
        in_specs=(
            [pl.BlockSpec((R, DC), lambda r: (r, 0))] * 4
            + [
                pl.BlockSpec((R, DC), lambda r: (r, 0)),
                pl.BlockSpec((512, 512), lambda r: (0, 0)),
                pl.BlockSpec((1, 512), lambda r: (0, 0)),
                pl.BlockSpec((512, 2 * DC), lambda r: (0, 0)),
            ]
        ),
        out_specs=[pl.BlockSpec((R, DC), lambda r: (r, 0))] * 2,
        out_shape=[jax.ShapeDtypeStruct((N, DC), jnp.float32)] * 2,
    )(*v_chunks, deg16, w3, b3_2d, wc)


def _finish_body(v0_ref, v1_ref, deg_ref, bmu_ref, blv_ref, mu_ref, lv_ref):
    d = _dis(deg_ref[...])
    mu_ref[...] = v0_ref[...] * d + bmu_ref[0]
    lv_ref[...] = v1_ref[...] * d + blv_ref[0]


def _finish(v_chunks, deg16, bmu2d, blv2d):
    return pl.pallas_call(
        _finish_body,
        grid=(N // R,),
        in_specs=[
            pl.BlockSpec((R, DC), lambda r: (r, 0)),
            pl.BlockSpec((R, DC), lambda r: (r, 0)),
            pl.BlockSpec((R, DC), lambda r: (r, 0)),
            pl.BlockSpec((1, DC), lambda r: (0, 0)),
            pl.BlockSpec((1, DC), lambda r: (0, 0)),
        ],
        out_specs=[pl.BlockSpec((R, DC), lambda r: (r, 0))] * 2,
        out_shape=[jax.ShapeDtypeStruct((N, DC), jnp.float32)] * 2,
    )(*v_chunks, deg16, bmu2d, blv2d)


def kernel(x, edge_index, W1, b1, W2, b2, W3, b3, W_mu, b_mu, W_lv, b_lv):
    src = edge_index[0].astype(jnp.int32)
    dst = edge_index[1].astype(jnp.int32)
    pad = jnp.arange(E2 - E, dtype=jnp.int32)
    src_p = jnp.concatenate([src, pad % N])
    dst_p = jnp.concatenate([dst, N + (pad % 8)])

    src3 = src_p.reshape(16, NB, 128)
    dst3 = dst_p.reshape(16, NB, 128)
    dst3deg = dst3

    zeros16 = jnp.zeros((N, DC), jnp.float32)
    ones16 = jnp.ones((128, DC), jnp.float32)
    deg16 = _make_deg_kernel()(dst3deg, zeros16, ones16)

    agg2 = _make_agg_kernel(2)
    agg4 = _make_agg_kernel(4)
    u0 = _scale_x(x, deg16)
    v1 = agg2(u0[0], u0[1], src3, dst3)
    u1 = _gcn_mid(v1, deg16, W1, b1.reshape(1, 512))
    v2 = agg4(*u1, src3, dst3)
    u2 = _gcn_mid(v2, deg16, W2, b2.reshape(1, 512))
    v3 = agg4(*u2, src3, dst3)
    wc = jnp.concatenate([W_mu, W_lv], axis=1)
    u3 = _gcn_final(v3, deg16, W3, b3.reshape(1, 512), wc)
    v4 = agg2(u3[0], u3[1], src3, dst3)
    mu, lv = _finish(v4, deg16, b_mu.reshape(1, DC), b_lv.reshape(1, DC))
    return (mu, lv)


# speedup vs baseline: 1.2315x; 1.0221x over previous
import functools

import jax
import jax.numpy as jnp
from jax import lax
from jax.experimental import pallas as pl
from jax.experimental.pallas import tpu as pltpu
from jax.experimental.pallas import tpu_sc as plsc

N = 10000
E = 160000
E2 = 163840
EPT = E2 // 16
NB = EPT // 128
RPT = 624
NC = 2
DC = 128
R = 1000

_HIGH = jax.lax.Precision.HIGHEST


def _mesh():
    return plsc.VectorSubcoreMesh(core_axis_name="c", subcore_axis_name="s")


def _make_deg_kernel():
    DEPTH = 8
    HBD = NB // 2

    @functools.partial(
        pl.kernel,
        mesh=_mesh(),
        out_type=[jax.ShapeDtypeStruct((N, DC), jnp.float32)] * 2,
        scratch_types=[
            pltpu.VMEM((HBD, 128), jnp.int32),
            pltpu.VMEM((128, DC), jnp.float32),
            pltpu.VMEM_SHARED((N + 8, DC), jnp.float32),
            pltpu.SemaphoreType.DMA,
        ],
    )
    def deg_kernel(dst_hbm, zeros_hbm, ones_hbm, out0, out1, didx, ones_v,
                   acc, sem):
        core = lax.axis_index("c")
        sub = lax.axis_index("s")

        pltpu.sync_copy(zeros_hbm.at[pl.ds(sub * RPT, RPT)],
                        acc.at[pl.ds(sub * RPT, RPT)])

        @pl.when(sub == 0)
        def _initt():
            pltpu.sync_copy(zeros_hbm.at[pl.ds(16 * RPT, N - 16 * RPT)],
                            acc.at[pl.ds(16 * RPT, N - 16 * RPT)])

        pltpu.sync_copy(ones_hbm, ones_v)
        pltpu.sync_copy(dst_hbm.at[sub, pl.ds(core * HBD, HBD)], didx)
        plsc.subcore_barrier()

        for r in range(DEPTH):
            pltpu.async_copy(ones_v, acc.at[didx.at[r]], sem, add=True)

        def body(j, carry):
            pltpu.make_async_copy(
                zeros_hbm.at[pl.ds(0, 128)], ones_v, sem).wait()
            pltpu.async_copy(
                ones_v, acc.at[didx.at[j + DEPTH]], sem, add=True)
            return carry

        lax.fori_loop(0, HBD - DEPTH, body, 0)
        for r in range(DEPTH):
            pltpu.make_async_copy(
                zeros_hbm.at[pl.ds(0, 128)], ones_v, sem).wait()
        plsc.subcore_barrier()

        for c, o in enumerate((out0, out1)):
            def _wout(o=o):
                pltpu.sync_copy(acc.at[pl.ds(sub * RPT, RPT)],
                                o.at[pl.ds(sub * RPT, RPT)])

                @pl.when(sub == 0)
                def _tail():
                    pltpu.sync_copy(acc.at[pl.ds(16 * RPT, N - 16 * RPT)],
                                    o.at[pl.ds(16 * RPT, N - 16 * RPT)])

            pl.when(core == c)(_wout)

    return deg_kernel


def _make_agg_kernel(nchunk):
    outs = [jax.ShapeDtypeStruct((N, DC), jnp.float32)] * nchunk

    HB = NB // 2

    @functools.partial(
        pl.kernel,
        mesh=_mesh(),
        out_type=outs,
        scratch_types=[
            pltpu.VMEM((HB, 128), jnp.int32),
            pltpu.VMEM((HB, 128), jnp.int32),
            pltpu.VMEM((128, DC), jnp.float32),
            pltpu.VMEM((128, DC), jnp.float32),
            pltpu.VMEM_SHARED((N + 8, DC), jnp.float32),
            pltpu.SemaphoreType.DMA,
            pltpu.SemaphoreType.DMA,
        ],
    )
    def agg_kernel(*refs):
        u = refs[:nchunk]
        src_hbm, dst_hbm = refs[nchunk], refs[nchunk + 1]
        out = refs[nchunk + 2:2 * nchunk + 2]
        sidx, didx, rows0, rows1, acc, sem0, sem1 = refs[2 * nchunk + 2:]
        core = lax.axis_index("c")
        sub = lax.axis_index("s")

        for ci in range(nchunk):
            def _chunk(ci=ci):
                pltpu.sync_copy(u[ci].at[pl.ds(sub * RPT, RPT)],
                                acc.at[pl.ds(sub * RPT, RPT)])

                @pl.when(sub == 0)
                def _init():
                    pltpu.sync_copy(u[ci].at[pl.ds(16 * RPT, N - 16 * RPT)],
                                    acc.at[pl.ds(16 * RPT, N - 16 * RPT)])

                plsc.subcore_barrier()

                for h in range(2):
                    pltpu.sync_copy(src_hbm.at[sub, pl.ds(h * HB, HB)], sidx)
                    pltpu.sync_copy(dst_hbm.at[sub, pl.ds(h * HB, HB)], didx)

                    pltpu.async_copy(u[ci].at[sidx.at[0]], rows0, sem0)

                    def body(j, carry):
                        b0 = 2 * j
                        pltpu.make_async_copy(
                            u[ci].at[pl.ds(0, 128)], rows0, sem0).wait()
                        pltpu.async_copy(
                            u[ci].at[sidx.at[b0 + 1]], rows1, sem1)
                        pltpu.sync_copy(rows0, acc.at[didx.at[b0]], add=True)

                        @pl.when(b0 + 2 < HB)
                        def _next():
                            pltpu.async_copy(
                                u[ci].at[sidx.at[b0 + 2]], rows0, sem0)

                        pltpu.make_async_copy(
                            u[ci].at[pl.ds(0, 128)], rows1, sem1).wait()
                        pltpu.sync_copy(
                            rows1, acc.at[didx.at[b0 + 1]], add=True)
                        return carry

                    lax.fori_loop(0, HB // 2, body, 0)
                plsc.subcore_barrier()
                pltpu.sync_copy(acc.at[pl.ds(sub * RPT, RPT)],
                                out[ci].at[pl.ds(sub * RPT, RPT)])

                @pl.when(sub == 0)
                def _tail():
                    pltpu.sync_copy(acc.at[pl.ds(16 * RPT, N - 16 * RPT)],
                                    out[ci].at[pl.ds(16 * RPT, N - 16 * RPT)])

                plsc.subcore_barrier()

            pl.when(core == (ci % NC))(_chunk)

    return agg_kernel


def _scale_x_body(x_ref, da_ref, db_ref, o0_ref, o1_ref, dis_ref):
    dis = jax.lax.rsqrt(da_ref[...] + db_ref[...] + 1.0)
    dis_ref[...] = dis
    u = x_ref[...] * dis[:, :1]
    o0_ref[...] = u[:, :DC]
    o1_ref[...] = u[:, DC:]


def _scale_x(x, dega, degb):
    return pl.pallas_call(
        _scale_x_body,
        grid=(N // R,),
        in_specs=[
            pl.BlockSpec((R, 2 * DC), lambda r: (r, 0)),
            pl.BlockSpec((R, DC), lambda r: (r, 0)),
            pl.BlockSpec((R, DC), lambda r: (r, 0)),
        ],
        out_specs=[pl.BlockSpec((R, DC), lambda r: (r, 0))] * 3,
        out_shape=[jax.ShapeDtypeStruct((N, DC), jnp.float32)] * 3,
    )(x, dega, degb)


def _gcn_mid_body(nc_in, *refs):
    v = refs[:nc_in]
    dis_ref, w_ref, b_ref = refs[nc_in:nc_in + 3]
    outs = refs[nc_in + 3:]
    d = dis_ref[:, :1]
    vv = jnp.concatenate([r[...] for r in v], axis=1) * d
    h = jnp.dot(vv, w_ref[...], precision=_HIGH) + b_ref[0]
    h = jnp.maximum(h, 0.0) * d
    for k, o in enumerate(outs):
        o[...] = h[:, k * DC:(k + 1) * DC]


def _gcn_mid(v_chunks, deg16, w, b2d):
    nc_in = len(v_chunks)
    k_dim = nc_in * DC
    return pl.pallas_call(
        functools.partial(_gcn_mid_body, nc_in),
        grid=(N // R,),
        in_specs=(
            [pl.BlockSpec((R, DC), lambda r: (r, 0))] * nc_in
            + [
                pl.BlockSpec((R, DC), lambda r: (r, 0)),
                pl.BlockSpec((k_dim, 512), lambda r: (0, 0)),
                pl.BlockSpec((1, 512), lambda r: (0, 0)),
            ]
        ),
        out_specs=[pl.BlockSpec((R, DC), lambda r: (r, 0))] * 4,
        out_shape=[jax.ShapeDtypeStruct((N, DC), jnp.float32)] * 4,
    )(*v_chunks, deg16, w, b2d)


def _gcn_final_body(*refs):
    v = refs[:4]
    dis_ref, w3_ref, b3_ref, wc_ref, o0_ref, o1_ref = refs[4:]
    d = dis_ref[:, :1]
    vv = jnp.concatenate([r[...] for r in v], axis=1) * d
    h = jnp.dot(vv, w3_ref[...], precision=_HIGH) + b3_ref[0]
    h = jnp.maximum(h, 0.0)
    z = jnp.dot(h, wc_ref[...], precision=_HIGH) * d
    o0_ref[...] = z[:, :DC]
    o1_ref[...] = z[:, DC:]


def _gcn_final(v_chunks, deg16, w3, b3_2d, wc):
    return pl.pallas_call(
        _gcn_final_body,
        grid=(N // R,),
        in_specs=(
            [pl.BlockSpec((R, DC), lambda r: (r, 0))] * 4
            + [
                pl.BlockSpec((R, DC), lambda r: (r, 0)),
                pl.BlockSpec((512, 512), lambda r: (0, 0)),
                pl.BlockSpec((1, 512), lambda r: (0, 0)),
                pl.BlockSpec((512, 2 * DC), lambda r: (0, 0)),
            ]
        ),
        out_specs=[pl.BlockSpec((R, DC), lambda r: (r, 0))] * 2,
        out_shape=[jax.ShapeDtypeStruct((N, DC), jnp.float32)] * 2,
    )(*v_chunks, deg16, w3, b3_2d, wc)


def _finish_body(v0_ref, v1_ref, dis_ref, bmu_ref, blv_ref, mu_ref, lv_ref):
    d = dis_ref[:, :1]
    mu_ref[...] = v0_ref[...] * d + bmu_ref[0]
    lv_ref[...] = v1_ref[...] * d + blv_ref[0]


def _finish(v_chunks, deg16, bmu2d, blv2d):
    return pl.pallas_call(
        _finish_body,
        grid=(N // R,),
        in_specs=[
            pl.BlockSpec((R, DC), lambda r: (r, 0)),
            pl.BlockSpec((R, DC), lambda r: (r, 0)),
            pl.BlockSpec((R, DC), lambda r: (r, 0)),
            pl.BlockSpec((1, DC), lambda r: (0, 0)),
            pl.BlockSpec((1, DC), lambda r: (0, 0)),
        ],
        out_specs=[pl.BlockSpec((R, DC), lambda r: (r, 0))] * 2,
        out_shape=[jax.ShapeDtypeStruct((N, DC), jnp.float32)] * 2,
    )(*v_chunks, deg16, bmu2d, blv2d)


def kernel(x, edge_index, W1, b1, W2, b2, W3, b3, W_mu, b_mu, W_lv, b_lv):
    src = edge_index[0].astype(jnp.int32)
    dst = edge_index[1].astype(jnp.int32)
    pad = jnp.arange(E2 - E, dtype=jnp.int32)
    src_p = jnp.concatenate([src, pad % N])
    dst_p = jnp.concatenate([dst, N + (pad % 8)])

    src3 = src_p.reshape(16, NB, 128)
    dst3 = dst_p.reshape(16, NB, 128)
    dst3deg = dst3

    zeros16 = jnp.zeros((N, DC), jnp.float32)
    ones16 = jnp.ones((128, DC), jnp.float32)
    dega, degb = _make_deg_kernel()(dst3deg, zeros16, ones16)

    agg2 = _make_agg_kernel(2)
    agg4 = _make_agg_kernel(4)
    u0a, u0b, dis128 = _scale_x(x, dega, degb)
    v1 = agg2(u0a, u0b, src3, dst3)
    u1 = _gcn_mid(v1, dis128, W1, b1.reshape(1, 512))
    v2 = agg4(*u1, src3, dst3)
    u2 = _gcn_mid(v2, dis128, W2, b2.reshape(1, 512))
    v3 = agg4(*u2, src3, dst3)
    wc = jnp.concatenate([W_mu, W_lv], axis=1)
    u3 = _gcn_final(v3, dis128, W3, b3.reshape(1, 512), wc)
    v4 = agg2(u3[0], u3[1], src3, dst3)
    mu, lv = _finish(v4, dis128, b_mu.reshape(1, DC), b_lv.reshape(1, DC))
    return (mu, lv)


# speedup vs baseline: 1.2336x; 1.0017x over previous
import functools

import jax
import jax.numpy as jnp
from jax import lax
from jax.experimental import pallas as pl
from jax.experimental.pallas import tpu as pltpu
from jax.experimental.pallas import tpu_sc as plsc

N = 10000
E = 160000
E2 = 163840
EPT = E2 // 16
NB = EPT // 128
RPT = 624
DR = 640
NC = 2
DC = 128
R = 1000

_HIGH = jax.lax.Precision.HIGHEST


def _mesh():
    return plsc.VectorSubcoreMesh(core_axis_name="c", subcore_axis_name="s")


def _make_deg_kernel():
    DEPTH = 8
    HBD = NB // 2

    @functools.partial(
        pl.kernel,
        mesh=_mesh(),
        out_type=[jax.ShapeDtypeStruct((N, DC), jnp.float32)] * 2,
        scratch_types=[
            pltpu.VMEM((HBD, 128), jnp.int32),
            pltpu.VMEM((128, DC), jnp.float32),
            pltpu.VMEM_SHARED((N + DR, DC), jnp.float32),
            pltpu.SemaphoreType.DMA,
        ],
    )
    def deg_kernel(dst_hbm, zeros_hbm, ones_hbm, out0, out1, didx, ones_v,
                   acc, sem):
        core = lax.axis_index("c")
        sub = lax.axis_index("s")

        pltpu.sync_copy(zeros_hbm.at[pl.ds(sub * RPT, RPT)],
                        acc.at[pl.ds(sub * RPT, RPT)])

        @pl.when(sub == 0)
        def _initt():
            pltpu.sync_copy(zeros_hbm.at[pl.ds(16 * RPT, N - 16 * RPT)],
                            acc.at[pl.ds(16 * RPT, N - 16 * RPT)])

        pltpu.sync_copy(ones_hbm, ones_v)
        pltpu.sync_copy(dst_hbm.at[sub, pl.ds(core * HBD, HBD)], didx)
        plsc.subcore_barrier()

        for r in range(DEPTH):
            pltpu.async_copy(ones_v, acc.at[didx.at[r]], sem, add=True)

        def body(j, carry):
            pltpu.make_async_copy(
                zeros_hbm.at[pl.ds(0, 128)], ones_v, sem).wait()
            pltpu.async_copy(
                ones_v, acc.at[didx.at[j + DEPTH]], sem, add=True)
            return carry

        lax.fori_loop(0, HBD - DEPTH, body, 0)
        for r in range(DEPTH):
            pltpu.make_async_copy(
                zeros_hbm.at[pl.ds(0, 128)], ones_v, sem).wait()
        plsc.subcore_barrier()

        for c, o in enumerate((out0, out1)):
            def _wout(o=o):
                pltpu.sync_copy(acc.at[pl.ds(sub * RPT, RPT)],
                                o.at[pl.ds(sub * RPT, RPT)])

                @pl.when(sub == 0)
                def _tail():
                    pltpu.sync_copy(acc.at[pl.ds(16 * RPT, N - 16 * RPT)],
                                    o.at[pl.ds(16 * RPT, N - 16 * RPT)])

            pl.when(core == c)(_wout)

    return deg_kernel


def _make_agg_kernel(nchunk):
    outs = [jax.ShapeDtypeStruct((N, DC), jnp.float32)] * nchunk

    HB = NB // 2

    @functools.partial(
        pl.kernel,
        mesh=_mesh(),
        out_type=outs,
        scratch_types=[
            pltpu.VMEM((HB, 128), jnp.int32),
            pltpu.VMEM((HB, 128), jnp.int32),
            pltpu.VMEM((128, DC), jnp.float32),
            pltpu.VMEM((128, DC), jnp.float32),
            pltpu.VMEM_SHARED((N + DR, DC), jnp.float32),
            pltpu.SemaphoreType.DMA,
            pltpu.SemaphoreType.DMA,
        ],
    )
    def agg_kernel(*refs):
        u = refs[:nchunk]
        src_hbm, dst_hbm = refs[nchunk], refs[nchunk + 1]
        out = refs[nchunk + 2:2 * nchunk + 2]
        sidx, didx, rows0, rows1, acc, sem0, sem1 = refs[2 * nchunk + 2:]
        core = lax.axis_index("c")
        sub = lax.axis_index("s")

        for ci in range(nchunk):
            def _chunk(ci=ci):
                pltpu.sync_copy(u[ci].at[pl.ds(sub * RPT, RPT)],
                                acc.at[pl.ds(sub * RPT, RPT)])

                @pl.when(sub == 0)
                def _init():
                    pltpu.sync_copy(u[ci].at[pl.ds(16 * RPT, N - 16 * RPT)],
                                    acc.at[pl.ds(16 * RPT, N - 16 * RPT)])

                plsc.subcore_barrier()

                for h in range(2):
                    pltpu.sync_copy(src_hbm.at[sub, pl.ds(h * HB, HB)], sidx)
                    pltpu.sync_copy(dst_hbm.at[sub, pl.ds(h * HB, HB)], didx)

                    pltpu.async_copy(u[ci].at[sidx.at[0]], rows0, sem0)

                    def body(j, carry):
                        b0 = 2 * j
                        pltpu.make_async_copy(
                            u[ci].at[pl.ds(0, 128)], rows0, sem0).wait()
                        pltpu.async_copy(
                            u[ci].at[sidx.at[b0 + 1]], rows1, sem1)
                        pltpu.sync_copy(rows0, acc.at[didx.at[b0]], add=True)

                        @pl.when(b0 + 2 < HB)
                        def _next():
                            pltpu.async_copy(
                                u[ci].at[sidx.at[b0 + 2]], rows0, sem0)

                        pltpu.make_async_copy(
                            u[ci].at[pl.ds(0, 128)], rows1, sem1).wait()
                        pltpu.sync_copy(
                            rows1, acc.at[didx.at[b0 + 1]], add=True)
                        return carry

                    lax.fori_loop(0, HB // 2, body, 0)
                plsc.subcore_barrier()
                pltpu.sync_copy(acc.at[pl.ds(sub * RPT, RPT)],
                                out[ci].at[pl.ds(sub * RPT, RPT)])

                @pl.when(sub == 0)
                def _tail():
                    pltpu.sync_copy(acc.at[pl.ds(16 * RPT, N - 16 * RPT)],
                                    out[ci].at[pl.ds(16 * RPT, N - 16 * RPT)])

                plsc.subcore_barrier()

            pl.when(core == (ci % NC))(_chunk)

    return agg_kernel


def _scale_x_body(x_ref, da_ref, db_ref, o0_ref, o1_ref, dis_ref):
    dis = jax.lax.rsqrt(da_ref[...] + db_ref[...] + 1.0)
    dis_ref[...] = dis
    u = x_ref[...] * dis[:, :1]
    o0_ref[...] = u[:, :DC]
    o1_ref[...] = u[:, DC:]


def _scale_x(x, dega, degb):
    return pl.pallas_call(
        _scale_x_body,
        grid=(N // R,),
        in_specs=[
            pl.BlockSpec((R, 2 * DC), lambda r: (r, 0)),
            pl.BlockSpec((R, DC), lambda r: (r, 0)),
            pl.BlockSpec((R, DC), lambda r: (r, 0)),
        ],
        out_specs=[pl.BlockSpec((R, DC), lambda r: (r, 0))] * 3,
        out_shape=[jax.ShapeDtypeStruct((N, DC), jnp.float32)] * 3,
    )(x, dega, degb)


def _gcn_mid_body(nc_in, *refs):
    v = refs[:nc_in]
    dis_ref, w_ref, b_ref = refs[nc_in:nc_in + 3]
    outs = refs[nc_in + 3:]
    d = dis_ref[:, :1]
    vv = jnp.concatenate([r[...] for r in v], axis=1) * d
    h = jnp.dot(vv, w_ref[...], precision=_HIGH) + b_ref[0]
    h = jnp.maximum(h, 0.0) * d
    for k, o in enumerate(outs):
        o[...] = h[:, k * DC:(k + 1) * DC]


def _gcn_mid(v_chunks, deg16, w, b2d):
    nc_in = len(v_chunks)
    k_dim = nc_in * DC
    return pl.pallas_call(
        functools.partial(_gcn_mid_body, nc_in),
        grid=(N // R,),
        in_specs=(
            [pl.BlockSpec((R, DC), lambda r: (r, 0))] * nc_in
            + [
                pl.BlockSpec((R, DC), lambda r: (r, 0)),
                pl.BlockSpec((k_dim, 512), lambda r: (0, 0)),
                pl.BlockSpec((1, 512), lambda r: (0, 0)),
            ]
        ),
        out_specs=[pl.BlockSpec((R, DC), lambda r: (r, 0))] * 4,
        out_shape=[jax.ShapeDtypeStruct((N, DC), jnp.float32)] * 4,
    )(*v_chunks, deg16, w, b2d)


def _gcn_final_body(*refs):
    v = refs[:4]
    dis_ref, w3_ref, b3_ref, wc_ref, o0_ref, o1_ref = refs[4:]
    d = dis_ref[:, :1]
    vv = jnp.concatenate([r[...] for r in v], axis=1) * d
    h = jnp.dot(vv, w3_ref[...], precision=_HIGH) + b3_ref[0]
    h = jnp.maximum(h, 0.0)
    z = jnp.dot(h, wc_ref[...], precision=_HIGH) * d
    o0_ref[...] = z[:, :DC]
    o1_ref[...] = z[:, DC:]


def _gcn_final(v_chunks, deg16, w3, b3_2d, wc):
    return pl.pallas_call(
        _gcn_final_body,
        grid=(N // R,),
        in_specs=(
            [pl.BlockSpec((R, DC), lambda r: (r, 0))] * 4
            + [
                pl.BlockSpec((R, DC), lambda r: (r, 0)),
                pl.BlockSpec((512, 512), lambda r: (0, 0)),
                pl.BlockSpec((1, 512), lambda r: (0, 0)),
                pl.BlockSpec((512, 2 * DC), lambda r: (0, 0)),
            ]
        ),
        out_specs=[pl.BlockSpec((R, DC), lambda r: (r, 0))] * 2,
        out_shape=[jax.ShapeDtypeStruct((N, DC), jnp.float32)] * 2,
    )(*v_chunks, deg16, w3, b3_2d, wc)


def _finish_body(v0_ref, v1_ref, dis_ref, bmu_ref, blv_ref, mu_ref, lv_ref):
    d = dis_ref[:, :1]
    mu_ref[...] = v0_ref[...] * d + bmu_ref[0]
    lv_ref[...] = v1_ref[...] * d + blv_ref[0]


def _finish(v_chunks, deg16, bmu2d, blv2d):
    return pl.pallas_call(
        _finish_body,
        grid=(N // R,),
        in_specs=[
            pl.BlockSpec((R, DC), lambda r: (r, 0)),
            pl.BlockSpec((R, DC), lambda r: (r, 0)),
            pl.BlockSpec((R, DC), lambda r: (r, 0)),
            pl.BlockSpec((1, DC), lambda r: (0, 0)),
            pl.BlockSpec((1, DC), lambda r: (0, 0)),
        ],
        out_specs=[pl.BlockSpec((R, DC), lambda r: (r, 0))] * 2,
        out_shape=[jax.ShapeDtypeStruct((N, DC), jnp.float32)] * 2,
    )(*v_chunks, deg16, bmu2d, blv2d)


def kernel(x, edge_index, W1, b1, W2, b2, W3, b3, W_mu, b_mu, W_lv, b_lv):
    src = edge_index[0].astype(jnp.int32)
    dst = edge_index[1].astype(jnp.int32)
    ppt = EPT - E // 16
    pad = jnp.arange(16 * ppt, dtype=jnp.int32)
    src_pad = (pad % N).reshape(16, ppt)
    dst_pad = (N + pad % DR).reshape(16, ppt)
    src3 = jnp.concatenate([src.reshape(16, E // 16), src_pad], axis=1)
    dst3 = jnp.concatenate([dst.reshape(16, E // 16), dst_pad], axis=1)
    src3 = src3.reshape(16, NB, 128)
    dst3 = dst3.reshape(16, NB, 128)
    dst3deg = dst3

    zeros16 = jnp.zeros((N, DC), jnp.float32)
    ones16 = jnp.ones((128, DC), jnp.float32)
    dega, degb = _make_deg_kernel()(dst3deg, zeros16, ones16)

    agg2 = _make_agg_kernel(2)
    agg4 = _make_agg_kernel(4)
    u0a, u0b, dis128 = _scale_x(x, dega, degb)
    v1 = agg2(u0a, u0b, src3, dst3)
    u1 = _gcn_mid(v1, dis128, W1, b1.reshape(1, 512))
    v2 = agg4(*u1, src3, dst3)
    u2 = _gcn_mid(v2, dis128, W2, b2.reshape(1, 512))
    v3 = agg4(*u2, src3, dst3)
    wc = jnp.concatenate([W_mu, W_lv], axis=1)
    u3 = _gcn_final(v3, dis128, W3, b3.reshape(1, 512), wc)
    v4 = agg2(u3[0], u3[1], src3, dst3)
    mu, lv = _finish(v4, dis128, b_mu.reshape(1, DC), b_lv.reshape(1, DC))
    return (mu, lv)
